# Initial kernel scaffold; baseline (speedup 1.0000x reference)
import jax
import jax.numpy as jnp
from jax.experimental import pallas as pl


def kernel(x, edge_index, W1, att_src1, att_dst1, bias1, W2, att_src2, att_dst2, bias2):
    raise NotImplementedError("write your pallas kernel here")


# speedup vs baseline: 43.5610x; 43.5610x over previous
import jax
import jax.numpy as jnp
from jax import lax
from jax.experimental import pallas as pl
from jax.experimental.pallas import tpu as pltpu
from jax.experimental.pallas import tpu_sc as plsc

N = 10000
D = 128
E = 320000
ET = E + N
NC, NS, L = 2, 16, 16
NW = NC * NS
CH = 128
KCH = -(-ET // (NW * CH))
EPW = KCH * CH
ETP = NW * EPW
NPAD = 10112
RPT = NPAD // NS
NEG = -1e30

f32 = jnp.float32
i32 = jnp.int32


def _tc1_body(x_ref, w_ref, asw_ref, adw_ref, h_ref, as_ref, ad_ref, m_ref):
    h = jnp.dot(x_ref[...], w_ref[...], preferred_element_type=f32)
    h_ref[:N, :] = h
    h_ref[N:, :] = jnp.zeros((NPAD - N, D), f32)
    s = jnp.sum(h * asw_ref[...][None, :], axis=1)
    d = jnp.sum(h * adw_ref[...][None, :], axis=1)
    as_ref[:N] = s
    as_ref[N:] = jnp.full((NPAD - N,), NEG, f32)
    ad_ref[:N] = d
    ad_ref[N:] = jnp.full((NPAD - N,), NEG, f32)
    m = jnp.max(s) + jnp.max(d)
    m = jnp.maximum(m, 0.2 * m)
    m_ref[...] = jnp.full((L,), m, f32)


def _tc2_body(acc_ref, accd_ref, b1_ref, w2_ref, asw_ref, adw_ref,
              h2_ref, as_ref, ad_ref, m_ref):
    den = jnp.sum(accd_ref[:, :N], axis=0) + 1e-16
    num = acc_ref[0, :N, :] + acc_ref[1, :N, :]
    out1 = num / den[:, None] + b1_ref[...][None, :]
    out1 = jnp.maximum(out1, 0.0)
    h2 = jnp.dot(out1, w2_ref[...], preferred_element_type=f32)
    h2_ref[...] = h2
    s = jnp.sum(h2 * asw_ref[...][None, :], axis=1)
    d = jnp.sum(h2 * adw_ref[...][None, :], axis=1)
    as_ref[:N] = s
    as_ref[N:] = jnp.full((NPAD - N,), NEG, f32)
    ad_ref[:N] = d
    ad_ref[N:] = jnp.full((NPAD - N,), NEG, f32)
    m = jnp.max(s) + jnp.max(d)
    m = jnp.maximum(m, 0.2 * m)
    m_ref[...] = jnp.full((L,), m, f32)


def _tc3_body(accd_ref, inv_ref):
    den = jnp.sum(accd_ref[...], axis=0) + 1e-16
    inv_ref[...] = 1.0 / den


def _tc4_body(accw_ref, h2_ref, b2_ref, out_ref):
    w = jnp.sum(accw_ref[:, :N], axis=0)
    out = jnp.dot(w[None, :], h2_ref[...], preferred_element_type=f32)
    out_ref[...] = out[0] * (1.0 / N) + b2_ref[...]


_MESH = plsc.VectorSubcoreMesh(core_axis_name="c", subcore_axis_name="s",
                               num_cores=NC, num_subcores=NS)
_CPARAMS = pltpu.CompilerParams(needs_layout_passes=False)


def _zero_vmem_2d(buf, nrows, ncols):
    z16 = jnp.zeros((L,), f32)

    @pl.loop(0, nrows)
    def _r(r):
        @pl.loop(0, ncols // L)
        def _c(g):
            buf[r, pl.ds(g * L, L)] = z16


def _zero_vmem_1d(buf, n):
    z16 = jnp.zeros((L,), f32)

    @pl.loop(0, n // L)
    def _i(i):
        buf[pl.ds(i * L, L)] = z16


def _scp_body(as_hbm, ad_hbm, m_hbm, src_hbm, dst_hbm,
              p_out, den_out,
              srcv, dstv, asv, adv, mv, pch, den):
    c = lax.axis_index("c")
    s = lax.axis_index("s")
    w = s * NC + c

    pltpu.sync_copy(src_hbm.at[w], srcv)
    pltpu.sync_copy(dst_hbm.at[w], dstv)
    pltpu.sync_copy(as_hbm, asv)
    pltpu.sync_copy(ad_hbm, adv)
    pltpu.sync_copy(m_hbm, mv)

    _zero_vmem_1d(den, NPAD)

    mvec = mv[...]
    lane = lax.iota(i32, L)
    masks = [lane == jnp.full((L,), l, i32) for l in range(L)]

    @pl.loop(0, KCH)
    def _chunk(j):
        @pl.loop(0, CH // L)
        def _grp(g):
            si = srcv[j, pl.ds(g * L, L)]
            di = dstv[j, pl.ds(g * L, L)]
            t = plsc.load_gather(asv, [si]) + plsc.load_gather(adv, [di])
            e = jnp.maximum(t, 0.2 * t)
            p = jnp.exp(e - mvec)
            pch[j, pl.ds(g * L, L)] = p
            for l in range(L):
                plsc.addupdate_scatter(den, [di], p, mask=masks[l])

    pltpu.sync_copy(pch, p_out.at[w])
    pltpu.sync_copy(den, den_out.at[w])


def _sch_body(h_hbm, p_hbm, src_hbm, dst_hbm,
              acc_out,
              srcv, dstv, pv, rows, acc, sem):
    c = lax.axis_index("c")
    s = lax.axis_index("s")
    w = s * NC + c
    base = s * RPT

    _zero_vmem_2d(rows, CH, D)
    nfull = RPT // CH

    @pl.loop(0, nfull)
    def _b(b):
        pltpu.sync_copy(rows, acc.at[pl.ds(base + b * CH, CH)])

    tail = RPT - nfull * CH
    if tail:
        pltpu.sync_copy(rows.at[pl.ds(0, tail)],
                        acc.at[pl.ds(base + nfull * CH, tail)])
    pl.delay(3000)
    plsc.subcore_barrier()

    @pl.loop(0, KCH)
    def _chunk(j):
        pltpu.sync_copy(src_hbm.at[w].at[j], srcv.at[0])
        pltpu.sync_copy(dst_hbm.at[w].at[j], dstv.at[0])
        pltpu.sync_copy(p_hbm.at[w].at[j], pv.at[0])
        pltpu.async_copy(h_hbm.at[srcv.at[0]], rows, sem).wait()

        @pl.loop(0, CH // L)
        def _grp(g):
            p = pv[0, pl.ds(g * L, L)]
            for r16 in range(L):
                ps = p[r16]
                r = g * L + r16
                for cg in range(D // L):
                    sl = pl.ds(cg * L, L)
                    rows[r, sl] = rows[r, sl] * ps

        pltpu.sync_copy(rows, acc.at[dstv.at[0]], add=True)

    plsc.subcore_barrier()
    pl.delay(3000)

    @pl.loop(0, nfull)
    def _d(b):
        sl = pl.ds(base + b * CH, CH)
        pltpu.sync_copy(acc.at[sl], rows)
        pltpu.sync_copy(rows, acc_out.at[c].at[sl])

    if tail:
        sl = pl.ds(base + nfull * CH, tail)
        pltpu.sync_copy(acc.at[sl], rows.at[pl.ds(0, tail)])
        pltpu.sync_copy(rows.at[pl.ds(0, tail)], acc_out.at[c].at[sl])


def _scw_body(p_hbm, inv_hbm, src_hbm, dst_hbm,
              w_out,
              srcv, dstv, invv, pch, wacc):
    c = lax.axis_index("c")
    s = lax.axis_index("s")
    w = s * NC + c

    pltpu.sync_copy(src_hbm.at[w], srcv)
    pltpu.sync_copy(dst_hbm.at[w], dstv)
    pltpu.sync_copy(inv_hbm, invv)
    pltpu.sync_copy(p_hbm.at[w], pch)

    _zero_vmem_1d(wacc, NPAD)

    lane = lax.iota(i32, L)
    masks = [lane == jnp.full((L,), l, i32) for l in range(L)]

    @pl.loop(0, KCH)
    def _chunk(j):
        @pl.loop(0, CH // L)
        def _grp(g):
            si = srcv[j, pl.ds(g * L, L)]
            di = dstv[j, pl.ds(g * L, L)]
            alpha = pch[j, pl.ds(g * L, L)] * plsc.load_gather(invv, [di])
            for l in range(L):
                plsc.addupdate_scatter(wacc, [si], alpha, mask=masks[l])

    pltpu.sync_copy(wacc, w_out.at[w])


def _sd(shape, dtype=f32):
    return jax.ShapeDtypeStruct(shape, dtype)


def _make_scp():
    return pl.kernel(
        _scp_body,
        out_type=[_sd((NW, KCH, CH)), _sd((NW, NPAD))],
        mesh=_MESH,
        compiler_params=_CPARAMS,
        scratch_types=[
            pltpu.VMEM((KCH, CH), i32),
            pltpu.VMEM((KCH, CH), i32),
            pltpu.VMEM((NPAD,), f32),
            pltpu.VMEM((NPAD,), f32),
            pltpu.VMEM((L,), f32),
            pltpu.VMEM((KCH, CH), f32),
            pltpu.VMEM((NPAD,), f32),
        ],
    )


def kernel(x, edge_index, W1, att_src1, att_dst1, bias1,
           W2, att_src2, att_dst2, bias2):
    loop_idx = jnp.arange(N, dtype=i32)
    pad_idx = N + (jnp.arange(ETP - ET, dtype=i32) % L)
    src = jnp.concatenate([edge_index[0].astype(i32), loop_idx, pad_idx])
    dst = jnp.concatenate([edge_index[1].astype(i32), loop_idx, pad_idx])
    src = src.reshape(NW, KCH, CH)
    dst = dst.reshape(NW, KCH, CH)

    h1e, as1e, ad1e, m1 = pl.pallas_call(
        _tc1_body,
        out_shape=[_sd((NPAD, D)), _sd((NPAD,)), _sd((NPAD,)), _sd((L,))],
    )(x, W1, att_src1, att_dst1)

    p1, den1 = _make_scp()(as1e, ad1e, m1, src, dst)

    sch = pl.kernel(
        _sch_body,
        out_type=_sd((NC, NPAD, D)),
        mesh=_MESH,
        compiler_params=_CPARAMS,
        scratch_types=[
            pltpu.VMEM((1, CH), i32),
            pltpu.VMEM((1, CH), i32),
            pltpu.VMEM((1, CH), f32),
            pltpu.VMEM((CH, D), f32),
            pltpu.VMEM_SHARED((NPAD, D), f32),
            pltpu.SemaphoreType.DMA,
        ],
    )
    acc1 = sch(h1e, p1, src, dst)

    h2, as2e, ad2e, m2 = pl.pallas_call(
        _tc2_body,
        out_shape=[_sd((N, D)), _sd((NPAD,)), _sd((NPAD,)), _sd((L,))],
    )(acc1, den1, bias1, W2, att_src2, att_dst2)

    p2, den2 = _make_scp()(as2e, ad2e, m2, src, dst)

    inv2 = pl.pallas_call(_tc3_body, out_shape=_sd((NPAD,)))(den2)

    scw = pl.kernel(
        _scw_body,
        out_type=_sd((NW, NPAD)),
        mesh=_MESH,
        compiler_params=_CPARAMS,
        scratch_types=[
            pltpu.VMEM((KCH, CH), i32),
            pltpu.VMEM((KCH, CH), i32),
            pltpu.VMEM((NPAD,), f32),
            pltpu.VMEM((KCH, CH), f32),
            pltpu.VMEM((NPAD,), f32),
        ],
    )
    accw = scw(p2, inv2, src, dst)

    out = pl.pallas_call(_tc4_body, out_shape=_sd((D,)))(accw, h2, bias2)
    return out


# speedup vs baseline: 55.2823x; 1.2691x over previous
import jax
import jax.numpy as jnp
from jax import lax
from jax.experimental import pallas as pl
from jax.experimental.pallas import tpu as pltpu
from jax.experimental.pallas import tpu_sc as plsc

N = 10000
D = 128
E = 320000
ET = E + N
NC, NS, L = 2, 16, 16
NW = NC * NS
CH = 128
KCH = -(-ET // (NW * CH))
EPW = KCH * CH
ETP = NW * EPW
NPAD = 10112
RPT = NPAD // NS
NEG = -1e30

f32 = jnp.float32
i32 = jnp.int32


def _tc1_body(x_ref, w_ref, asw_ref, adw_ref, h_ref, as_ref, ad_ref, m_ref):
    h = jnp.dot(x_ref[...], w_ref[...], preferred_element_type=f32)
    h_ref[:N, :] = h
    h_ref[N:, :] = jnp.zeros((NPAD - N, D), f32)
    s = jnp.sum(h * asw_ref[...][None, :], axis=1)
    d = jnp.sum(h * adw_ref[...][None, :], axis=1)
    as_ref[:N] = s
    as_ref[N:] = jnp.full((NPAD - N,), NEG, f32)
    ad_ref[:N] = d
    ad_ref[N:] = jnp.full((NPAD - N,), NEG, f32)
    m = jnp.max(s) + jnp.max(d)
    m = jnp.maximum(m, 0.2 * m)
    m_ref[...] = jnp.full((L,), m, f32)


def _tc2_body(acc_ref, accd_ref, b1_ref, w2_ref, asw_ref, adw_ref,
              h2_ref, as_ref, ad_ref, m_ref):
    den = jnp.sum(accd_ref[:, :N], axis=0) + 1e-16
    num = acc_ref[0, :N, :] + acc_ref[1, :N, :]
    out1 = num / den[:, None] + b1_ref[...][None, :]
    out1 = jnp.maximum(out1, 0.0)
    h2 = jnp.dot(out1, w2_ref[...], preferred_element_type=f32)
    h2_ref[...] = h2
    s = jnp.sum(h2 * asw_ref[...][None, :], axis=1)
    d = jnp.sum(h2 * adw_ref[...][None, :], axis=1)
    as_ref[:N] = s
    as_ref[N:] = jnp.full((NPAD - N,), NEG, f32)
    ad_ref[:N] = d
    ad_ref[N:] = jnp.full((NPAD - N,), NEG, f32)
    m = jnp.max(s) + jnp.max(d)
    m = jnp.maximum(m, 0.2 * m)
    m_ref[...] = jnp.full((L,), m, f32)


def _tc3_body(accd_ref, inv_ref):
    den = jnp.sum(accd_ref[...], axis=0) + 1e-16
    inv_ref[...] = 1.0 / den


def _tc4_body(accw_ref, h2_ref, b2_ref, out_ref):
    w = jnp.sum(accw_ref[:, :N], axis=0)
    out = jnp.dot(w[None, :], h2_ref[...], preferred_element_type=f32)
    out_ref[...] = out[0] * (1.0 / N) + b2_ref[...]


_MESH = plsc.VectorSubcoreMesh(core_axis_name="c", subcore_axis_name="s",
                               num_cores=NC, num_subcores=NS)
_CPARAMS = pltpu.CompilerParams(needs_layout_passes=False)


def _zero_vmem_2d(buf, nrows, ncols):
    z16 = jnp.zeros((L,), f32)

    @pl.loop(0, nrows)
    def _r(r):
        @pl.loop(0, ncols // L)
        def _c(g):
            buf[r, pl.ds(g * L, L)] = z16


def _zero_vmem_1d(buf, n):
    z16 = jnp.zeros((L,), f32)

    @pl.loop(0, n // L)
    def _i(i):
        buf[pl.ds(i * L, L)] = z16


def _scp_body(as_hbm, ad_hbm, m_hbm, src_hbm, dst_hbm,
              p_out, den_out,
              srcv, dstv, asv, adv, mv, pch, den):
    c = lax.axis_index("c")
    s = lax.axis_index("s")
    w = s * NC + c

    pltpu.sync_copy(src_hbm.at[w], srcv)
    pltpu.sync_copy(dst_hbm.at[w], dstv)
    pltpu.sync_copy(as_hbm, asv)
    pltpu.sync_copy(ad_hbm, adv)
    pltpu.sync_copy(m_hbm, mv)

    _zero_vmem_1d(den, NPAD)

    mvec = mv[...]
    lane = lax.iota(i32, L)
    masks = [lane == jnp.full((L,), l, i32) for l in range(L)]

    @pl.loop(0, KCH)
    def _chunk(j):
        @pl.loop(0, CH // L)
        def _grp(g):
            si = srcv[j, pl.ds(g * L, L)]
            di = dstv[j, pl.ds(g * L, L)]
            t = plsc.load_gather(asv, [si]) + plsc.load_gather(adv, [di])
            e = jnp.maximum(t, 0.2 * t)
            p = jnp.exp(e - mvec)
            pch[j, pl.ds(g * L, L)] = p
            for l in range(L):
                plsc.addupdate_scatter(den, [di], p, mask=masks[l])

    pltpu.sync_copy(pch, p_out.at[w])
    pltpu.sync_copy(den, den_out.at[w])


def _sch_body(h_hbm, p_hbm, src_hbm, dst_hbm,
              acc_out,
              srcv, dstv, pv, rows, acc, sem):
    c = lax.axis_index("c")
    s = lax.axis_index("s")
    w = s * NC + c
    base = s * RPT

    _zero_vmem_2d(rows, CH, D)
    nfull = RPT // CH

    @pl.loop(0, nfull)
    def _b(b):
        pltpu.sync_copy(rows, acc.at[pl.ds(base + b * CH, CH)])

    tail = RPT - nfull * CH
    if tail:
        pltpu.sync_copy(rows.at[pl.ds(0, tail)],
                        acc.at[pl.ds(base + nfull * CH, tail)])
    pl.delay(3000)
    plsc.subcore_barrier()

    pltpu.sync_copy(src_hbm.at[w], srcv)
    pltpu.sync_copy(dst_hbm.at[w], dstv)
    pltpu.sync_copy(p_hbm.at[w], pv)

    @pl.loop(0, KCH)
    def _chunk(j):
        pltpu.async_copy(h_hbm.at[srcv.at[j]], rows, sem).wait()

        @pl.loop(0, CH // L)
        def _grp(g):
            p = pv[j, pl.ds(g * L, L)]
            for r16 in range(L):
                ps = p[r16]
                r = g * L + r16
                for cg in range(D // L):
                    sl = pl.ds(cg * L, L)
                    rows[r, sl] = rows[r, sl] * ps

        pltpu.sync_copy(rows, acc.at[dstv.at[j]], add=True)

    plsc.subcore_barrier()
    pl.delay(3000)

    @pl.loop(0, nfull)
    def _d(b):
        sl = pl.ds(base + b * CH, CH)
        pltpu.sync_copy(acc.at[sl], rows)
        pltpu.sync_copy(rows, acc_out.at[c].at[sl])

    if tail:
        sl = pl.ds(base + nfull * CH, tail)
        pltpu.sync_copy(acc.at[sl], rows.at[pl.ds(0, tail)])
        pltpu.sync_copy(rows.at[pl.ds(0, tail)], acc_out.at[c].at[sl])


def _scw_body(p_hbm, inv_hbm, src_hbm, dst_hbm,
              w_out,
              srcv, dstv, invv, pch, wacc):
    c = lax.axis_index("c")
    s = lax.axis_index("s")
    w = s * NC + c

    pltpu.sync_copy(src_hbm.at[w], srcv)
    pltpu.sync_copy(dst_hbm.at[w], dstv)
    pltpu.sync_copy(inv_hbm, invv)
    pltpu.sync_copy(p_hbm.at[w], pch)

    _zero_vmem_1d(wacc, NPAD)

    lane = lax.iota(i32, L)
    masks = [lane == jnp.full((L,), l, i32) for l in range(L)]

    @pl.loop(0, KCH)
    def _chunk(j):
        @pl.loop(0, CH // L)
        def _grp(g):
            si = srcv[j, pl.ds(g * L, L)]
            di = dstv[j, pl.ds(g * L, L)]
            alpha = pch[j, pl.ds(g * L, L)] * plsc.load_gather(invv, [di])
            for l in range(L):
                plsc.addupdate_scatter(wacc, [si], alpha, mask=masks[l])

    pltpu.sync_copy(wacc, w_out.at[w])


def _sd(shape, dtype=f32):
    return jax.ShapeDtypeStruct(shape, dtype)


def _make_scp():
    return pl.kernel(
        _scp_body,
        out_type=[_sd((NW, KCH, CH)), _sd((NW, NPAD))],
        mesh=_MESH,
        compiler_params=_CPARAMS,
        scratch_types=[
            pltpu.VMEM((KCH, CH), i32),
            pltpu.VMEM((KCH, CH), i32),
            pltpu.VMEM((NPAD,), f32),
            pltpu.VMEM((NPAD,), f32),
            pltpu.VMEM((L,), f32),
            pltpu.VMEM((KCH, CH), f32),
            pltpu.VMEM((NPAD,), f32),
        ],
    )


def kernel(x, edge_index, W1, att_src1, att_dst1, bias1,
           W2, att_src2, att_dst2, bias2):
    loop_idx = jnp.arange(N, dtype=i32)
    pad_idx = N + (jnp.arange(ETP - ET, dtype=i32) % L)
    src = jnp.concatenate([edge_index[0].astype(i32), loop_idx, pad_idx])
    dst = jnp.concatenate([edge_index[1].astype(i32), loop_idx, pad_idx])
    src = src.reshape(NW, KCH, CH)
    dst = dst.reshape(NW, KCH, CH)

    h1e, as1e, ad1e, m1 = pl.pallas_call(
        _tc1_body,
        out_shape=[_sd((NPAD, D)), _sd((NPAD,)), _sd((NPAD,)), _sd((L,))],
    )(x, W1, att_src1, att_dst1)

    p1, den1 = _make_scp()(as1e, ad1e, m1, src, dst)

    sch = pl.kernel(
        _sch_body,
        out_type=_sd((NC, NPAD, D)),
        mesh=_MESH,
        compiler_params=_CPARAMS,
        scratch_types=[
            pltpu.VMEM((KCH, CH), i32),
            pltpu.VMEM((KCH, CH), i32),
            pltpu.VMEM((KCH, CH), f32),
            pltpu.VMEM((CH, D), f32),
            pltpu.VMEM_SHARED((NPAD, D), f32),
            pltpu.SemaphoreType.DMA,
        ],
    )
    acc1 = sch(h1e, p1, src, dst)

    h2, as2e, ad2e, m2 = pl.pallas_call(
        _tc2_body,
        out_shape=[_sd((N, D)), _sd((NPAD,)), _sd((NPAD,)), _sd((L,))],
    )(acc1, den1, bias1, W2, att_src2, att_dst2)

    p2, den2 = _make_scp()(as2e, ad2e, m2, src, dst)

    inv2 = pl.pallas_call(_tc3_body, out_shape=_sd((NPAD,)))(den2)

    scw = pl.kernel(
        _scw_body,
        out_type=_sd((NW, NPAD)),
        mesh=_MESH,
        compiler_params=_CPARAMS,
        scratch_types=[
            pltpu.VMEM((KCH, CH), i32),
            pltpu.VMEM((KCH, CH), i32),
            pltpu.VMEM((NPAD,), f32),
            pltpu.VMEM((KCH, CH), f32),
            pltpu.VMEM((NPAD,), f32),
        ],
    )
    accw = scw(p2, inv2, src, dst)

    out = pl.pallas_call(_tc4_body, out_shape=_sd((D,)))(accw, h2, bias2)
    return out
